# baseline (speedup 1.0000x reference)
import functools

import jax
import jax.numpy as jnp
from jax import lax
from jax.experimental import pallas as pl
from jax.experimental.pallas import tpu as pltpu
from jax.experimental.pallas import tpu_sc as plsc

_EMB = 128
_WIN = 64
_LANES = 16
_REGS_PER_ROW = _EMB // _LANES
_CH = 128
_NB = 4
_LEAD = 2


def _run(flat_idx, token_table, pos_table):
    N = flat_idx.shape[0]
    V, D = token_table.shape

    info = plsc.get_sparse_core_info()
    NC, NS = info.num_cores, info.num_subcores
    NW = NC * NS
    n_per_w = N // NW
    n_ch = n_per_w // _CH
    rounds = n_ch // _NB

    mesh = plsc.VectorSubcoreMesh(core_axis_name="c", subcore_axis_name="s")

    @functools.partial(
        pl.kernel,
        mesh=mesh,
        out_type=jax.ShapeDtypeStruct((N, D), jnp.float32),
        scratch_types=(
            [pltpu.VMEM((n_per_w,), jnp.int32),
             pltpu.VMEM((_WIN, D), jnp.float32),
             pltpu.VMEM_SHARED((V, D), jnp.float32)]
            + [pltpu.VMEM((_CH, D), jnp.float32) for _ in range(_NB)]
            + [pltpu.SemaphoreType.DMA for _ in range(2 * _NB)]
        ),
    )
    def k(idx_hbm, tok_hbm, pos_hbm, out_hbm, idx_all, pos_v, tab_sh,
          *bufs_and_sems):
        rows = list(bufs_and_sems[:_NB])
        gsem = list(bufs_and_sems[_NB:2 * _NB])
        osem = list(bufs_and_sems[2 * _NB:])

        sid = lax.axis_index("s")
        wid = sid * NC + lax.axis_index("c")
        base = wid * n_per_w
        v_per_s = V // NS
        pltpu.sync_copy(tok_hbm.at[pl.ds(sid * v_per_s, v_per_s)],
                        tab_sh.at[pl.ds(sid * v_per_s, v_per_s)])
        pltpu.sync_copy(pos_hbm, pos_v)
        pltpu.sync_copy(idx_hbm.at[pl.ds(base, n_per_w)], idx_all)
        plsc.subcore_barrier()

        def gather_add_start(lci, b):
            src = tab_sh.at[idx_all.at[pl.ds(lci * _CH, _CH)]]
            pltpu.async_copy(src, rows[b], gsem[b], add=True)

        def gather_wait(lci, b):
            src = tab_sh.at[idx_all.at[pl.ds(lci * _CH, _CH)]]
            pltpu.make_async_copy(src, rows[b], gsem[b]).wait()

        def out_copy(lci, b):
            return pltpu.make_async_copy(
                rows[b], out_hbm.at[pl.ds(base + lci * _CH, _CH)], osem[b])

        def pos_fill(b):
            rows_b = rows[b]

            def w_body(wi, _):
                for u in range(2):
                    w = wi * 2 + u
                    for kk in range(_REGS_PER_ROW):
                        sl = pl.ds(kk * _LANES, _LANES)
                        pv = pos_v[w, sl]
                        for r in range(_CH // _WIN):
                            rows_b[r * _WIN + w, sl] = pv
                return 0

            lax.fori_loop(0, _WIN // 2, w_body, 0)

        for b in range(_NB):
            pos_fill(b)
            gather_add_start(b, b)

        def round_body(i, _):
            for b in range(_NB):
                lci = i * _NB + b
                gather_wait(lci, b)
                out_copy(lci, b).start()

                bn = (b + _LEAD) % _NB
                cond = (i >= 1) if b < _LEAD else (i < rounds - 1)

                @pl.when(cond)
                def _():
                    out_copy(lci + _LEAD - _NB, bn).wait()
                    pos_fill(bn)
                    gather_add_start(lci + _LEAD, bn)

            return 0

        lax.fori_loop(0, rounds, round_body, 0)
        for b in range(_NB):
            out_copy((rounds - 1) * _NB + b, b).wait()

    return k(flat_idx, token_table, pos_table)


def kernel(input, token_table, pos_table):
    B, W = input.shape
    D = token_table.shape[1]
    flat_idx = input.reshape(B * W).astype(jnp.int32)
    out = _run(flat_idx, token_table, pos_table)
    return out.reshape(B, W, D)
